# baseline (speedup 1.0000x reference)
import functools
import jax
import jax.numpy as jnp
from jax import lax
from jax.experimental import pallas as pl
from jax.experimental.pallas import tpu as pltpu
from jax.experimental.pallas import tpu_sc as plsc

H = 32
W = 32
D = 256
HW = H * W
NCORES = 2
NSUB = 16
CPS = (2 * D) // NSUB
BPC = 32 // NCORES
CHUNK = CPS * HW
BATCH_F = 2 * D * HW


def _sc_body(col_hbm, row_hbm, out_hbm, table_v, buf_v, sem, dsem):
    cid = lax.axis_index("c")
    sid = lax.axis_index("s")
    is_col = sid < (NSUB // 2)

    @pl.when(is_col)
    def _():
        pltpu.async_copy(col_hbm.at[pl.ds(0, W * D)], table_v, sem).wait()

    @pl.when(jnp.logical_not(is_col))
    def _():
        pltpu.async_copy(row_hbm.at[pl.ds(0, H * D)], table_v, sem).wait()

    lanes = lax.broadcasted_iota(jnp.int32, (16,), 0)
    col0 = (sid * CPS) % D

    def build_col(cc, _):
        cidx = jnp.full((16,), col0 + cc, dtype=jnp.int32)
        v_lo = plsc.load_gather(table_v, [lanes * D + cidx])
        v_hi = plsc.load_gather(table_v, [(lanes + 16) * D + cidx])
        base = cc * HW
        for rep in range(W):
            buf_v[pl.ds(base + rep * W, 16)] = v_lo
            buf_v[pl.ds(base + rep * W + 16, 16)] = v_hi
        return 0

    def build_row(cc, _):
        base = cc * HW
        for h in range(H):
            v = plsc.load_gather(
                table_v, [jnp.full((16,), h * D + col0 + cc, jnp.int32)])
            buf_v[pl.ds(base + h * W, 16)] = v
            buf_v[pl.ds(base + h * W + 16, 16)] = v
        return 0

    @pl.when(is_col)
    def _():
        lax.fori_loop(0, CPS, build_col, 0)

    @pl.when(jnp.logical_not(is_col))
    def _():
        lax.fori_loop(0, CPS, build_row, 0)

    foff = sid * CHUNK
    copies = []
    for j in range(BPC):
        b = cid * BPC + j
        copies.append(
            pltpu.async_copy(buf_v, out_hbm.at[b, pl.ds(foff, CHUNK)], dsem))
    for cp in copies:
        cp.wait()


def kernel(x, row_embed, col_embed):
    batch = x.shape[0]
    mesh = plsc.VectorSubcoreMesh(core_axis_name="c", subcore_axis_name="s")
    run = functools.partial(
        pl.kernel,
        out_type=jax.ShapeDtypeStruct((batch, BATCH_F), jnp.float32),
        mesh=mesh,
        scratch_types=[
            pltpu.VMEM((W * D,), jnp.float32),
            pltpu.VMEM((CHUNK,), jnp.float32),
            pltpu.SemaphoreType.DMA,
            pltpu.SemaphoreType.DMA,
        ],
        compiler_params=pltpu.CompilerParams(needs_layout_passes=False),
    )(_sc_body)
    out2 = run(col_embed.reshape(-1), row_embed.reshape(-1))
    return out2.reshape(batch, 2 * D, H, W)


# speedup vs baseline: 3.6199x; 3.6199x over previous
import functools
import jax
import jax.numpy as jnp
from jax import lax
from jax.experimental import pallas as pl
from jax.experimental.pallas import tpu as pltpu
from jax.experimental.pallas import tpu_sc as plsc

H = 32
W = 32
D = 256
HW = H * W
NCORES = 2
NSUB = 16
CPS = (2 * D) // NSUB
BPC = 32 // NCORES
CHUNK = CPS * HW
BATCH_F = 2 * D * HW


def _sc_body(col_hbm, row_hbm, out_hbm, table_v, buf_v, sem, dsem):
    cid = lax.axis_index("c")
    sid = lax.axis_index("s")
    is_col = sid < (NSUB // 2)

    @pl.when(is_col)
    def _():
        pltpu.async_copy(col_hbm.at[pl.ds(0, W * D)], table_v, sem).wait()

    @pl.when(jnp.logical_not(is_col))
    def _():
        pltpu.async_copy(row_hbm.at[pl.ds(0, H * D)], table_v, sem).wait()

    lanes = lax.broadcasted_iota(jnp.int32, (16,), 0)
    col0 = (sid * CPS) % D

    def build_col(cc, _):
        cidx = jnp.full((16,), col0 + cc, dtype=jnp.int32)
        v_lo = plsc.load_gather(table_v, [lanes * D + cidx])
        v_hi = plsc.load_gather(table_v, [(lanes + 16) * D + cidx])
        for rep in range(W):
            buf_v[cc, pl.ds(rep * W, 16)] = v_lo
            buf_v[cc, pl.ds(rep * W + 16, 16)] = v_hi
        return 0

    def build_row(cc, _):
        for h in range(H):
            v = plsc.load_gather(
                table_v, [jnp.full((16,), h * D + col0 + cc, jnp.int32)])
            buf_v[cc, pl.ds(h * W, 16)] = v
            buf_v[cc, pl.ds(h * W + 16, 16)] = v
        return 0

    @pl.when(is_col)
    def _():
        lax.fori_loop(0, CPS, build_col, 0)

    @pl.when(jnp.logical_not(is_col))
    def _():
        lax.fori_loop(0, CPS, build_row, 0)

    c0 = sid * CPS
    copies = []
    for j in range(BPC):
        b = cid * BPC + j
        copies.append(
            pltpu.async_copy(buf_v, out_hbm.at[b, pl.ds(c0, CPS), :], dsem))
    for cp in copies:
        cp.wait()


def kernel(x, row_embed, col_embed):
    batch = x.shape[0]
    mesh = plsc.VectorSubcoreMesh(core_axis_name="c", subcore_axis_name="s")
    run = functools.partial(
        pl.kernel,
        out_type=jax.ShapeDtypeStruct((batch, 2 * D, HW), jnp.float32),
        mesh=mesh,
        scratch_types=[
            pltpu.VMEM((W * D,), jnp.float32),
            pltpu.VMEM((CPS, HW), jnp.float32),
            pltpu.SemaphoreType.DMA,
            pltpu.SemaphoreType.DMA,
        ],
        compiler_params=pltpu.CompilerParams(
            needs_layout_passes=False, use_tc_tiling_on_sc=True),
    )(_sc_body)
    out2 = run(col_embed.reshape(-1), row_embed.reshape(-1))
    return out2.reshape(batch, 2 * D, H, W)


# speedup vs baseline: 7.9193x; 2.1877x over previous
import functools
import jax
import jax.numpy as jnp
from jax import lax
from jax.experimental import pallas as pl
from jax.experimental.pallas import tpu as pltpu
from jax.experimental.pallas import tpu_sc as plsc

H = 32
W = 32
D = 256
NCORES = 2
NSUB = 16
HPS = H // NSUB
BPC = 32 // NCORES
L = 16


def _sc_body(col_hbm, row_hbm, out_hbm, col_v, row_v, buf_v, sem, dsem):
    cid = lax.axis_index("c")
    sid = lax.axis_index("s")
    h0 = sid * HPS

    cp1 = pltpu.async_copy(col_hbm.at[pl.ds(0, W), :], col_v, sem)
    cp2 = pltpu.async_copy(row_hbm.at[pl.ds(h0, HPS), :], row_v, sem)
    cp1.wait()
    cp2.wait()

    for hl in range(HPS):
        row_regs = [row_v[hl, pl.ds(k * L, L)] for k in range(D // L)]
        for w in range(W):
            for k in range(D // L):
                buf_v[hl, w, pl.ds(k * L, L)] = col_v[w, pl.ds(k * L, L)]
                buf_v[hl, w, pl.ds(D + k * L, L)] = row_regs[k]

    copies = []
    for j in range(BPC):
        b = cid * BPC + j
        copies.append(
            pltpu.async_copy(buf_v, out_hbm.at[b, pl.ds(h0, HPS), :, :], dsem))
    for cp in copies:
        cp.wait()


def kernel(x, row_embed, col_embed):
    batch = x.shape[0]
    mesh = plsc.VectorSubcoreMesh(core_axis_name="c", subcore_axis_name="s")
    run = functools.partial(
        pl.kernel,
        out_type=jax.ShapeDtypeStruct((batch, H, W, 2 * D), jnp.float32),
        mesh=mesh,
        scratch_types=[
            pltpu.VMEM((W, D), jnp.float32),
            pltpu.VMEM((HPS, D), jnp.float32),
            pltpu.VMEM((HPS, W, 2 * D), jnp.float32),
            pltpu.SemaphoreType.DMA,
            pltpu.SemaphoreType.DMA,
        ],
        compiler_params=pltpu.CompilerParams(
            needs_layout_passes=False, use_tc_tiling_on_sc=True),
    )(_sc_body)
    out_cm = run(col_embed, row_embed)
    return jnp.transpose(out_cm, (0, 3, 1, 2))
